# baseline (speedup 1.0000x reference)
import functools

import jax
import jax.numpy as jnp
from jax import lax
from jax.experimental import pallas as pl
from jax.experimental.pallas import tpu as pltpu
from jax.experimental.pallas import tpu_sc as plsc

B = 16384
D = 16
NC = 2
NS = 16
NW = NC * NS
BPW = B // NW
CH = 128
NCH = BPW // CH
NBLK = BPW // 16


@functools.cache
def _nmf_sc():
    mesh = plsc.VectorSubcoreMesh(core_axis_name="c", subcore_axis_name="s")

    @functools.partial(
        pl.kernel,
        mesh=mesh,
        out_type=(jax.ShapeDtypeStruct((B,), jnp.float32),
                  jax.ShapeDtypeStruct((NW, 16), jnp.float32)),
        scratch_types=[
            pltpu.VMEM((NCH, CH), jnp.int32),
            pltpu.VMEM((NCH, CH), jnp.int32),
            pltpu.VMEM((BPW, D), jnp.float32),
            pltpu.VMEM((BPW, D), jnp.float32),
            pltpu.VMEM((BPW, D), jnp.float32),
            pltpu.VMEM((BPW, D), jnp.float32),
            pltpu.VMEM((BPW,), jnp.float32),
            pltpu.VMEM((BPW,), jnp.float32),
            pltpu.VMEM((16,), jnp.float32),
            pltpu.VMEM((2 * D * D, D), jnp.float32),
            pltpu.VMEM((D, D), jnp.float32),
            pltpu.SemaphoreType.DMA,
        ],
        compiler_params=pltpu.CompilerParams(
            use_tc_tiling_on_sc=False, needs_layout_passes=False),
    )
    def nmf_sc(user_hbm, item_hbm, umlp_hbm, imlp_hbm, umf_hbm, imf_hbm,
               rat_hbm, w1_hbm, w2_hbm,
               tgt_hbm, lp_hbm,
               uidx, iidx, b0, b1, b2, b3, rbuf, tbuf, lbuf, wbv, w2v, sem):
        wid = lax.axis_index("s") * NC + lax.axis_index("c")
        pltpu.sync_copy(user_hbm.at[wid], uidx)
        pltpu.sync_copy(item_hbm.at[wid], iidx)
        pltpu.sync_copy(rat_hbm.at[wid], rbuf)
        pltpu.sync_copy(w1_hbm, wbv)
        pltpu.sync_copy(w2_hbm, w2v)
        copies = []
        for table, idx, buf in ((umlp_hbm, uidx, b0), (imlp_hbm, iidx, b1),
                                (umf_hbm, uidx, b2), (imf_hbm, iidx, b3)):
            for j in range(NCH):
                copies.append(pltpu.async_copy(
                    table.at[idx.at[j]], buf.at[pl.ds(j * CH, CH)], sem))
        for c in copies:
            c.wait()

        lane = lax.iota(jnp.int32, 16)

        def tanh_e(e):
            return (e - 1.0) / (e + 1.0)

        def rsqrt_nr(x):
            i = plsc.bitcast(x, jnp.int32)
            y = plsc.bitcast(0x5F3759DF - (i >> 1), jnp.float32)
            for _ in range(3):
                y = y * (1.5 - 0.5 * x * y * y)
            return y

        def block(bi, lacc):
            row = bi * 16 + lane
            zero = jnp.zeros((16,), jnp.float32)
            h = [zero] * D
            su = sv = suu = svv = suv = zero
            for d in range(D):
                col = jnp.full((16,), d, jnp.int32)
                cu = plsc.load_gather(b0, [row, col])
                ci = plsc.load_gather(b1, [row, col])
                xu = plsc.load_gather(b2, [row, col])
                xv = plsc.load_gather(b3, [row, col])
                for j in range(D):
                    h[j] = (h[j] + wbv[d * D + j, :] * cu
                            + wbv[D * D + d * D + j, :] * ci)
                su = su + xu
                sv = sv + xv
                suu = suu + xu * xu
                svv = svv + xv * xv
                suv = suv + xu * xv
            m = jnp.zeros((16,), jnp.float32)
            for j in range(D):
                m = m + w2v[j, :] * tanh_e(jnp.exp(h[j]))
            mlp = tanh_e(jnp.exp(m))
            ssu = suu - su * su * (1.0 / 16.0)
            ssv = svv - sv * sv * (1.0 / 16.0)
            duv = suv - su * sv * (1.0 / 16.0)
            nu = jnp.maximum(ssu * rsqrt_nr(ssu), 1e-12)
            nv = jnp.maximum(ssv * rsqrt_nr(ssv), 1e-12)
            mf = duv / (nu * nv)
            nmf = 0.5 * (mlp + mf)
            rn = rbuf[pl.ds(bi * 16, 16)] * 0.5 - 1.5
            diff = nmf - rn
            tbuf[pl.ds(bi * 16, 16)] = nmf * 2.0 + 3.0
            return lacc + diff * diff

        lacc = lax.fori_loop(0, NBLK, block, jnp.zeros((16,), jnp.float32))
        lbuf[...] = lacc * (1.0 / B)
        base = wid * BPW
        pltpu.sync_copy(tbuf, tgt_hbm.at[pl.ds(base, BPW)])
        pltpu.sync_copy(lbuf, lp_hbm.at[wid])

    return nmf_sc


def kernel(user, item, rating, U_mlp, I_mlp, U_mf, I_mf, W1, W2):
    user_r = user.astype(jnp.int32).reshape(NW, NCH, CH)
    item_r = item.astype(jnp.int32).reshape(NW, NCH, CH)
    rat_r = rating.reshape(NW, BPW)
    wu = (2.0 * W1[:, :D]).T.reshape(D * D, 1)
    wi = (2.0 * W1[:, D:]).T.reshape(D * D, 1)
    wb = jnp.broadcast_to(jnp.concatenate([wu, wi], axis=0), (2 * D * D, D))
    w2b = jnp.broadcast_to((2.0 * W2).reshape(D, 1), (D, D))
    tgt, lparts = _nmf_sc()(user_r, item_r, U_mlp, I_mlp, U_mf, I_mf,
                            rat_r, wb, w2b)
    return jnp.sum(lparts), tgt
